# baseline (speedup 1.0000x reference)
import dataclasses
import functools

import jax
import jax.numpy as jnp
from jax import lax
from jax.experimental import pallas as pl
from jax.experimental.pallas import tpu as pltpu
from jax.experimental.pallas import tpu_sc as plsc

N = 10000
E = 320000
D = 128

NC = 2
NS = 16
NW = NC * NS

CH = 32
NCH = 320
EPT = NCH * CH
E2 = NW * EPT
G = CH // 16
RING = 4
NPAD = 10240
ROWS_PT = NPAD // NS

MM_BLK = 1000
NB = N // MM_BLK
BN_BLK = 1024
NBP = NPAD // BN_BLK


def _matmul_body(x_ref, w_ref, o_ref):
    o_ref[...] = jnp.dot(x_ref[...], w_ref[...],
                         preferred_element_type=jnp.float32)


def _matmul(x, W):
    return pl.pallas_call(
        _matmul_body,
        grid=(NB,),
        in_specs=[
            pl.BlockSpec((MM_BLK, D), lambda i: (i, 0)),
            pl.BlockSpec((D, D), lambda i: (0, 0)),
        ],
        out_specs=pl.BlockSpec((MM_BLK, D), lambda i: (i, 0)),
        out_shape=jax.ShapeDtypeStruct((N, D), jnp.float32),
    )(x, W)


def _spmm_body(ps_hbm, meta_hbm, zeros_hbm, out_hbm,
               m0, m1, m2, m3, r0_, r1_, r2_, r3_,
               acc, msems, gsems, ssems):
    metas = (m0, m1, m2, m3)
    rows = (r0_, r1_, r2_, r3_)
    c = lax.axis_index("c")
    s = lax.axis_index("s")
    wid = c * NS + s

    def meta_dma(ci, b):
        return pltpu.make_async_copy(meta_hbm.at[wid, ci], metas[b],
                                     msems[b])

    def gather(ci, b):
        return pltpu.make_async_copy(ps_hbm.at[metas[b].at[0]], rows[b],
                                     gsems[b])

    def scatter_wait(b):
        pltpu.make_async_copy(rows[b], acc.at[metas[b].at[1]],
                              ssems[b]).wait()

    for t in (0, 1):
        meta_dma(t, t).start()
    rbase = s * ROWS_PT
    pltpu.sync_copy(zeros_hbm, acc.at[pl.ds(rbase, ROWS_PT)])
    plsc.subcore_barrier()
    meta_dma(0, 0).wait()
    gather(0, 0).start()

    @pl.loop(0, NCH, step=RING)
    def _outer(j):
        for b in range(RING):
            ci = j + b
            b1 = (b + 1) % RING
            b2 = (b + 2) % RING

            @pl.when(ci >= 2)
            def _():
                scatter_wait(b2)

            @pl.when(ci + 2 < NCH)
            def _():
                meta_dma(ci + 2, b2).start()

            @pl.when(ci + 1 < NCH)
            def _():
                meta_dma(ci + 1, b1).wait()
                gather(ci + 1, b1).start()

            gather(ci, b).wait()

            rv = rows[b]
            mv = metas[b]
            two_idx = jnp.full((16,), 2, jnp.int32)

            @pl.loop(0, CH, unroll=5)
            def _edge(e):
                bw = plsc.bitcast(
                    plsc.load_gather(mv, [two_idx,
                                          jnp.full((16,), e, jnp.int32)]),
                    jnp.float32)
                for k in range(D // 16):
                    sl = pl.ds(k * 16, 16)
                    rv[e, sl] = rv[e, sl] * bw

            pltpu.async_copy(rows[b], acc.at[metas[b].at[1]],
                             ssems[b], add=True)

    for t in (NCH - 2, NCH - 1):
        scatter_wait(t % RING)

    plsc.subcore_barrier()
    pltpu.sync_copy(acc.at[pl.ds(rbase, ROWS_PT)],
                    out_hbm.at[c, pl.ds(rbase, ROWS_PT)])


def _spmm(pre_sup, meta, zeros):
    mesh = plsc.VectorSubcoreMesh(core_axis_name="c", subcore_axis_name="s")
    cp = pltpu.CompilerParams()
    if "needs_layout_passes" in pltpu.CompilerParams.__dataclass_fields__:
        cp = dataclasses.replace(cp, needs_layout_passes=False)
    k = pl.kernel(
        _spmm_body,
        mesh=mesh,
        compiler_params=cp,
        out_type=jax.ShapeDtypeStruct((NC, NPAD, D), jnp.float32),
        scratch_types=(
            [pltpu.VMEM((3, CH), jnp.int32)] * RING
            + [pltpu.VMEM((CH, D), jnp.float32)] * RING
            + [
                pltpu.VMEM_SHARED((NPAD, D), jnp.float32),
                [pltpu.SemaphoreType.DMA] * RING,
                [pltpu.SemaphoreType.DMA] * RING,
                [pltpu.SemaphoreType.DMA] * RING,
            ]
        ),
    )
    return k(pre_sup, meta, zeros)


def _stats_body(p_ref, comb_ref, stat_ref):
    i = pl.program_id(0)
    blk = p_ref[0] + p_ref[1]
    comb_ref[...] = blk

    s0 = jnp.sum(blk, axis=0, keepdims=True)
    s1 = jnp.sum(blk * blk, axis=0, keepdims=True)
    upd = jnp.concatenate([s0, s1, jnp.zeros((6, D), jnp.float32)], axis=0)

    @pl.when(i == 0)
    def _():
        stat_ref[...] = jnp.zeros((8, D), jnp.float32)

    stat_ref[...] += upd


def _stats(partials):
    return pl.pallas_call(
        _stats_body,
        grid=(NBP,),
        in_specs=[pl.BlockSpec((NC, BN_BLK, D), lambda i: (0, i, 0))],
        out_specs=[
            pl.BlockSpec((BN_BLK, D), lambda i: (i, 0)),
            pl.BlockSpec((8, D), lambda i: (0, 0)),
        ],
        out_shape=[
            jax.ShapeDtypeStruct((NPAD, D), jnp.float32),
            jax.ShapeDtypeStruct((8, D), jnp.float32),
        ],
    )(partials)


def _norm_body(comb_ref, stat_ref, o_ref):
    mean = stat_ref[0:1, :] * (1.0 / N)
    var = stat_ref[1:2, :] * (1.0 / N) - mean * mean
    inv = lax.rsqrt(var + 0.001)
    o_ref[...] = jnp.maximum((comb_ref[...] - mean) * inv, 0.0)


def _norm(comb, stats):
    return pl.pallas_call(
        _norm_body,
        grid=(NBP,),
        in_specs=[
            pl.BlockSpec((BN_BLK, D), lambda i: (i, 0)),
            pl.BlockSpec((8, D), lambda i: (0, 0)),
        ],
        out_specs=pl.BlockSpec((BN_BLK, D), lambda i: (i, 0)),
        out_shape=jax.ShapeDtypeStruct((NPAD, D), jnp.float32),
    )(comb, stats)


def kernel(x, edge_index, edge_weight, W):
    dst = edge_index[0]
    src = edge_index[1]
    meta = jnp.stack(
        [src, dst, lax.bitcast_convert_type(edge_weight, jnp.int32)], axis=0)
    meta = jnp.pad(meta, ((0, 0), (0, E2 - E)))
    meta = meta.reshape(3, NW, NCH, CH).transpose(1, 2, 0, 3)
    zeros = jnp.zeros((ROWS_PT, D), jnp.float32)

    pre_sup = _matmul(x, W)
    partials = _spmm(pre_sup, meta, zeros)
    comb, stats = _stats(partials)
    return _norm(comb, stats)[:N]


# speedup vs baseline: 2.3803x; 2.3803x over previous
import dataclasses
import functools

import jax
import jax.numpy as jnp
from jax import lax
from jax.experimental import pallas as pl
from jax.experimental.pallas import tpu as pltpu
from jax.experimental.pallas import tpu_sc as plsc

N = 10000
E = 320000
D = 128

NC = 2
NS = 16
NW = NC * NS

CH = 50
NCH = 200
EPT = NCH * CH
E2 = NW * EPT
GS = 10
G = CH // GS
RING = 4
NPAD = 10240
ROWS_PT = NPAD // NS

MM_BLK = 1000
NB = N // MM_BLK
BN_BLK = 1024
NBP = NPAD // BN_BLK


def _matmul_body(x_ref, w_ref, o_ref):
    o_ref[...] = jnp.dot(x_ref[...], w_ref[...],
                         preferred_element_type=jnp.float32)


def _matmul(x, W):
    return pl.pallas_call(
        _matmul_body,
        grid=(NB,),
        in_specs=[
            pl.BlockSpec((MM_BLK, D), lambda i: (i, 0)),
            pl.BlockSpec((D, D), lambda i: (0, 0)),
        ],
        out_specs=pl.BlockSpec((MM_BLK, D), lambda i: (i, 0)),
        out_shape=jax.ShapeDtypeStruct((N, D), jnp.float32),
    )(x, W)


def _spmm_body(ps_hbm, meta_hbm, zeros_hbm, out_hbm,
               m0, m1, m2, m3, r0_, r1_, r2_, r3_,
               acc, msems, gsems, ssems):
    metas = (m0, m1, m2, m3)
    rows = (r0_, r1_, r2_, r3_)
    c = lax.axis_index("c")
    s = lax.axis_index("s")
    wid = c * NS + s

    def meta_dma(ci, b):
        return pltpu.make_async_copy(meta_hbm.at[wid, ci], metas[b],
                                     msems[b])

    def gather(ci, b):
        return pltpu.make_async_copy(ps_hbm.at[metas[b].at[0]], rows[b],
                                     gsems[b])

    def scatter_wait(b):
        pltpu.make_async_copy(rows[b], acc.at[metas[b].at[1]],
                              ssems[b]).wait()

    for t in (0, 1):
        meta_dma(t, t).start()
    rbase = s * ROWS_PT
    pltpu.sync_copy(zeros_hbm, acc.at[pl.ds(rbase, ROWS_PT)])
    plsc.subcore_barrier()
    meta_dma(0, 0).wait()
    gather(0, 0).start()

    @pl.loop(0, NCH, step=RING)
    def _outer(j):
        for b in range(RING):
            ci = j + b
            b1 = (b + 1) % RING
            b2 = (b + 2) % RING

            @pl.when(ci >= 2)
            def _():
                scatter_wait(b2)

            @pl.when(ci + 2 < NCH)
            def _():
                meta_dma(ci + 2, b2).start()

            @pl.when(ci + 1 < NCH)
            def _():
                meta_dma(ci + 1, b1).wait()
                gather(ci + 1, b1).start()

            gather(ci, b).wait()

            rv = rows[b]
            mv = metas[b]
            two_idx = jnp.full((16,), 2, jnp.int32)
            iota16 = lax.iota(jnp.int32, 16)

            @pl.loop(0, G)
            def _grp(g):
                lanes = jnp.minimum(iota16 + g * GS, CH - 1)
                wv = plsc.bitcast(plsc.load_gather(mv, [two_idx, lanes]),
                                  jnp.float32)
                for k in range(GS):
                    bw = jnp.full((16,), wv[k], jnp.float32)
                    for jj in range(D // 16):
                        sl = pl.ds(jj * 16, 16)
                        rv[g * GS + k, sl] = rv[g * GS + k, sl] * bw

            pltpu.async_copy(rows[b], acc.at[metas[b].at[1]],
                             ssems[b], add=True)

    for t in (NCH - 2, NCH - 1):
        scatter_wait(t % RING)

    plsc.subcore_barrier()
    pltpu.sync_copy(acc.at[pl.ds(rbase, ROWS_PT)],
                    out_hbm.at[c, pl.ds(rbase, ROWS_PT)])


def _spmm(pre_sup, meta, zeros):
    mesh = plsc.VectorSubcoreMesh(core_axis_name="c", subcore_axis_name="s")
    cp = pltpu.CompilerParams()
    if "needs_layout_passes" in pltpu.CompilerParams.__dataclass_fields__:
        cp = dataclasses.replace(cp, needs_layout_passes=False)
    k = pl.kernel(
        _spmm_body,
        mesh=mesh,
        compiler_params=cp,
        out_type=jax.ShapeDtypeStruct((NC, NPAD, D), jnp.float32),
        scratch_types=(
            [pltpu.VMEM((3, CH), jnp.int32)] * RING
            + [pltpu.VMEM((CH, D), jnp.float32)] * RING
            + [
                pltpu.VMEM_SHARED((NPAD, D), jnp.float32),
                [pltpu.SemaphoreType.DMA] * RING,
                [pltpu.SemaphoreType.DMA] * RING,
                [pltpu.SemaphoreType.DMA] * RING,
            ]
        ),
    )
    return k(pre_sup, meta, zeros)


def _stats_body(p_ref, comb_ref, stat_ref):
    i = pl.program_id(0)
    blk = p_ref[0] + p_ref[1]
    comb_ref[...] = blk

    s0 = jnp.sum(blk, axis=0, keepdims=True)
    s1 = jnp.sum(blk * blk, axis=0, keepdims=True)
    upd = jnp.concatenate([s0, s1, jnp.zeros((6, D), jnp.float32)], axis=0)

    @pl.when(i == 0)
    def _():
        stat_ref[...] = jnp.zeros((8, D), jnp.float32)

    stat_ref[...] += upd


def _stats(partials):
    return pl.pallas_call(
        _stats_body,
        grid=(NBP,),
        in_specs=[pl.BlockSpec((NC, BN_BLK, D), lambda i: (0, i, 0))],
        out_specs=[
            pl.BlockSpec((BN_BLK, D), lambda i: (i, 0)),
            pl.BlockSpec((8, D), lambda i: (0, 0)),
        ],
        out_shape=[
            jax.ShapeDtypeStruct((NPAD, D), jnp.float32),
            jax.ShapeDtypeStruct((8, D), jnp.float32),
        ],
    )(partials)


def _norm_body(comb_ref, stat_ref, o_ref):
    mean = stat_ref[0:1, :] * (1.0 / N)
    var = stat_ref[1:2, :] * (1.0 / N) - mean * mean
    inv = lax.rsqrt(var + 0.001)
    o_ref[...] = jnp.maximum((comb_ref[...] - mean) * inv, 0.0)


def _norm(comb, stats):
    return pl.pallas_call(
        _norm_body,
        grid=(NBP,),
        in_specs=[
            pl.BlockSpec((BN_BLK, D), lambda i: (i, 0)),
            pl.BlockSpec((8, D), lambda i: (0, 0)),
        ],
        out_specs=pl.BlockSpec((BN_BLK, D), lambda i: (i, 0)),
        out_shape=jax.ShapeDtypeStruct((NPAD, D), jnp.float32),
    )(comb, stats)


def kernel(x, edge_index, edge_weight, W):
    dst = edge_index[0]
    src = edge_index[1]
    meta = jnp.stack(
        [src, dst, lax.bitcast_convert_type(edge_weight, jnp.int32)], axis=0)
    meta = jnp.pad(meta, ((0, 0), (0, E2 - E)))
    meta = meta.reshape(3, NW, NCH, CH).transpose(1, 2, 0, 3)
    zeros = jnp.zeros((ROWS_PT, D), jnp.float32)

    pre_sup = _matmul(x, W)
    partials = _spmm(pre_sup, meta, zeros)
    comb, stats = _stats(partials)
    return _norm(comb, stats)[:N]


# speedup vs baseline: 2.7012x; 1.1348x over previous
import dataclasses
import functools

import jax
import jax.numpy as jnp
from jax import lax
from jax.experimental import pallas as pl
from jax.experimental.pallas import tpu as pltpu
from jax.experimental.pallas import tpu_sc as plsc

N = 10000
E = 320000
D = 128

NC = 2
NS = 16
NW = NC * NS

CH = 50
NCH = 200
EPT = NCH * CH
E2 = NW * EPT
GS = 10
G = CH // GS
RING = 4
NPAD = 10240
ROWS_PT = NPAD // NS

MM_BLK = 1000
NB = N // MM_BLK
BN_BLK = 1024
NBP = NPAD // BN_BLK


def _matmul_body(x_ref, w_ref, o_ref):
    o_ref[...] = jnp.dot(x_ref[...], w_ref[...],
                         preferred_element_type=jnp.float32)


def _matmul(x, W):
    return pl.pallas_call(
        _matmul_body,
        grid=(NB,),
        in_specs=[
            pl.BlockSpec((MM_BLK, D), lambda i: (i, 0)),
            pl.BlockSpec((D, D), lambda i: (0, 0)),
        ],
        out_specs=pl.BlockSpec((MM_BLK, D), lambda i: (i, 0)),
        out_shape=jax.ShapeDtypeStruct((N, D), jnp.float32),
    )(x, W)


def _spmm_body(ps_hbm, ei_hbm, ew_hbm, zeros_hbm, out_hbm,
               m0, m1, m2, m3, w0, w1, w2, w3, r0_, r1_, r2_, r3_,
               acc, msems, wsems, gsems, ssems):
    metas = (m0, m1, m2, m3)
    wrows = (w0, w1, w2, w3)
    rows = (r0_, r1_, r2_, r3_)
    c = lax.axis_index("c")
    s = lax.axis_index("s")
    wid = c * NS + s
    crow0 = wid * NCH

    def src_dma(ci, b):
        return pltpu.make_async_copy(ei_hbm.at[1, crow0 + ci],
                                     metas[b].at[0], msems[b])

    def dst_dma(ci, b):
        return pltpu.make_async_copy(ei_hbm.at[0, crow0 + ci],
                                     metas[b].at[1], msems[b])

    def w_dma(ci, b):
        return pltpu.make_async_copy(ew_hbm.at[crow0 + ci], wrows[b],
                                     wsems[b])

    def meta_start(ci, b):
        src_dma(ci, b).start()
        dst_dma(ci, b).start()
        w_dma(ci, b).start()

    def meta_wait_idx(ci, b):
        src_dma(ci, b).wait()
        dst_dma(ci, b).wait()

    def gather(ci, b):
        return pltpu.make_async_copy(ps_hbm.at[metas[b].at[0]], rows[b],
                                     gsems[b])

    def scatter_wait(b):
        pltpu.make_async_copy(rows[b], acc.at[metas[b].at[1]],
                              ssems[b]).wait()

    for t in (0, 1):
        meta_start(t, t)
    rbase = s * ROWS_PT
    pltpu.sync_copy(zeros_hbm, acc.at[pl.ds(rbase, ROWS_PT)])
    plsc.subcore_barrier()
    meta_wait_idx(0, 0)
    gather(0, 0).start()

    @pl.loop(0, NCH, step=RING)
    def _outer(j):
        for b in range(RING):
            ci = j + b
            b1 = (b + 1) % RING
            b2 = (b + 2) % RING

            @pl.when(ci >= 2)
            def _():
                scatter_wait(b2)

            @pl.when(ci + 2 < NCH)
            def _():
                meta_start(ci + 2, b2)

            @pl.when(ci + 1 < NCH)
            def _():
                meta_wait_idx(ci + 1, b1)
                gather(ci + 1, b1).start()

            gather(ci, b).wait()
            w_dma(ci, b).wait()

            rv = rows[b]
            wr = wrows[b]
            iota16 = lax.iota(jnp.int32, 16)

            @pl.loop(0, G)
            def _grp(g):
                lanes = jnp.minimum(iota16 + g * GS, CH - 1)
                wv = plsc.load_gather(wr, [lanes])
                for k in range(GS):
                    bw = jnp.full((16,), wv[k], jnp.float32)
                    for jj in range(D // 16):
                        sl = pl.ds(jj * 16, 16)
                        rv[g * GS + k, sl] = rv[g * GS + k, sl] * bw

            pltpu.async_copy(rows[b], acc.at[metas[b].at[1]],
                             ssems[b], add=True)

    for t in (NCH - 2, NCH - 1):
        scatter_wait(t % RING)

    plsc.subcore_barrier()
    pltpu.sync_copy(acc.at[pl.ds(rbase, ROWS_PT)],
                    out_hbm.at[c, pl.ds(rbase, ROWS_PT)])


def _spmm(pre_sup, ei, ew, zeros):
    mesh = plsc.VectorSubcoreMesh(core_axis_name="c", subcore_axis_name="s")
    cp = pltpu.CompilerParams()
    if "needs_layout_passes" in pltpu.CompilerParams.__dataclass_fields__:
        cp = dataclasses.replace(cp, needs_layout_passes=False)
    k = pl.kernel(
        _spmm_body,
        mesh=mesh,
        compiler_params=cp,
        out_type=jax.ShapeDtypeStruct((NC, NPAD, D), jnp.float32),
        scratch_types=(
            [pltpu.VMEM((2, CH), jnp.int32)] * RING
            + [pltpu.VMEM((CH,), jnp.float32)] * RING
            + [pltpu.VMEM((CH, D), jnp.float32)] * RING
            + [
                pltpu.VMEM_SHARED((NPAD, D), jnp.float32),
                [pltpu.SemaphoreType.DMA] * RING,
                [pltpu.SemaphoreType.DMA] * RING,
                [pltpu.SemaphoreType.DMA] * RING,
                [pltpu.SemaphoreType.DMA] * RING,
            ]
        ),
    )
    return k(pre_sup, ei, ew, zeros)


def _stats_body(p_ref, comb_ref, stat_ref):
    i = pl.program_id(0)
    blk = p_ref[0] + p_ref[1]
    comb_ref[...] = blk

    s0 = jnp.sum(blk, axis=0, keepdims=True)
    s1 = jnp.sum(blk * blk, axis=0, keepdims=True)
    upd = jnp.concatenate([s0, s1, jnp.zeros((6, D), jnp.float32)], axis=0)

    @pl.when(i == 0)
    def _():
        stat_ref[...] = jnp.zeros((8, D), jnp.float32)

    stat_ref[...] += upd


def _stats(partials):
    return pl.pallas_call(
        _stats_body,
        grid=(NBP,),
        in_specs=[pl.BlockSpec((NC, BN_BLK, D), lambda i: (0, i, 0))],
        out_specs=[
            pl.BlockSpec((BN_BLK, D), lambda i: (i, 0)),
            pl.BlockSpec((8, D), lambda i: (0, 0)),
        ],
        out_shape=[
            jax.ShapeDtypeStruct((NPAD, D), jnp.float32),
            jax.ShapeDtypeStruct((8, D), jnp.float32),
        ],
    )(partials)


def _norm_body(comb_ref, stat_ref, o_ref):
    mean = stat_ref[0:1, :] * (1.0 / N)
    var = stat_ref[1:2, :] * (1.0 / N) - mean * mean
    inv = lax.rsqrt(var + 0.001)
    o_ref[...] = jnp.maximum((comb_ref[...] - mean) * inv, 0.0)


def _norm(comb, stats):
    return pl.pallas_call(
        _norm_body,
        grid=(NBP,),
        in_specs=[
            pl.BlockSpec((BN_BLK, D), lambda i: (i, 0)),
            pl.BlockSpec((8, D), lambda i: (0, 0)),
        ],
        out_specs=pl.BlockSpec((BN_BLK, D), lambda i: (i, 0)),
        out_shape=jax.ShapeDtypeStruct((NPAD, D), jnp.float32),
    )(comb, stats)


def kernel(x, edge_index, edge_weight, W):
    ei = edge_index.reshape(2, NW * NCH, CH)
    ew = edge_weight.reshape(NW * NCH, CH)
    zeros = jnp.zeros((ROWS_PT, D), jnp.float32)

    pre_sup = _matmul(x, W)
    partials = _spmm(pre_sup, ei, ew, zeros)
    comb, stats = _stats(partials)
    return _norm(comb, stats)[:N]
